# Initial kernel scaffold; baseline (speedup 1.0000x reference)
import functools

import jax
import jax.numpy as jnp
from jax import lax
from jax.experimental import pallas as pl
from jax.experimental.pallas import tpu as pltpu
from jax.experimental.pallas import tpu_sc as plsc

NC = 2
NS = 16
NW = NC * NS
CH = 128


def _sc_segment_sum(src3, dst3, h, zeros, n_acc, rpt_zero, rpt_out, nch):
    n, d = h.shape
    mesh = plsc.VectorSubcoreMesh(
        core_axis_name="c", subcore_axis_name="s", num_cores=NC, num_subcores=NS
    )

    def body(src_hbm, dst_hbm, h_hbm, zeros_hbm, out_hbm,
             src_v, dst_v, buf0, buf1, acc, sem0, sem1):
        c = lax.axis_index("c")
        s = lax.axis_index("s")
        w = c * NS + s

        pltpu.sync_copy(src_hbm.at[w], src_v)
        pltpu.sync_copy(dst_hbm.at[w], dst_v)

        pltpu.async_copy(h_hbm.at[src_v.at[0]], buf0, sem0)
        pltpu.async_copy(h_hbm.at[src_v.at[1]], buf1, sem1)

        pltpu.sync_copy(zeros_hbm, acc.at[pl.ds(s * rpt_zero, rpt_zero)])
        plsc.subcore_barrier()

        bufs = (buf0, buf1)
        sems = (sem0, sem1)

        def chunk_pair(k, carry):
            for b in range(2):
                j = 2 * k + b
                pltpu.make_async_copy(h_hbm.at[src_v.at[j]], bufs[b], sems[b]).wait()
                pltpu.sync_copy(bufs[b], acc.at[dst_v.at[j]], add=True)
                pltpu.async_copy(h_hbm.at[src_v.at[j + 2]], bufs[b], sems[b])
            return carry

        lax.fori_loop(0, nch // 2 - 1, chunk_pair, 0)

        for b in range(2):
            j = nch - 2 + b
            pltpu.make_async_copy(h_hbm.at[src_v.at[j]], bufs[b], sems[b]).wait()
            pltpu.sync_copy(bufs[b], acc.at[dst_v.at[j]], add=True)

        plsc.subcore_barrier()
        pltpu.sync_copy(acc.at[pl.ds(s * rpt_out, rpt_out)],
                        out_hbm.at[c, pl.ds(s * rpt_out, rpt_out)])

    fn = pl.kernel(
        body,
        out_type=jax.ShapeDtypeStruct((NC, n, d), jnp.float32),
        mesh=mesh,
        scratch_types=[
            pltpu.VMEM((nch, CH), jnp.int32),
            pltpu.VMEM((nch, CH), jnp.int32),
            pltpu.VMEM((CH, d), jnp.float32),
            pltpu.VMEM((CH, d), jnp.float32),
            pltpu.VMEM_SHARED((n_acc, d), jnp.float32),
            pltpu.SemaphoreType.DMA,
            pltpu.SemaphoreType.DMA,
        ],
    )
    return fn(src3, dst3, h, zeros)


def _tc_body(h_ref, p_ref, ws_ref, wn_ref, bs_ref, bn_ref, g_ref, be_ref, out_ref):
    x = h_ref[...]
    p = p_ref[0] + p_ref[1]
    sh = jnp.dot(x, ws_ref[...], preferred_element_type=jnp.float32) + bs_ref[...]
    nh = jnp.dot(p, wn_ref[...], preferred_element_type=jnp.float32) + bn_ref[...]
    cat = jnp.concatenate([sh, nh], axis=1)
    mu = jnp.mean(cat, axis=1, keepdims=True)
    var = jnp.mean((cat - mu) * (cat - mu), axis=1, keepdims=True)
    out_ref[...] = (cat - mu) * lax.rsqrt(var + 1e-5) * g_ref[...] + be_ref[...]


def kernel(edge_index, h, W_self, b_self, W_neigh, b_neigh, gamma, beta):
    n, d = h.shape
    o = W_self.shape[1]
    e = edge_index.shape[1]

    nch = -(-e // (NW * CH))
    nch += nch % 2
    e_pad = NW * CH * nch
    dst = edge_index[0]
    src = edge_index[1]
    src_p = jnp.concatenate([src, jnp.zeros((e_pad - e,), jnp.int32)])
    dst_p = jnp.concatenate([dst, jnp.full((e_pad - e,), n, jnp.int32)])
    src3 = src_p.reshape(NW, nch, CH)
    dst3 = dst_p.reshape(NW, nch, CH)

    rpt_zero = -(-(n + 1) // NS)
    n_acc = NS * rpt_zero
    rpt_out = n // NS
    zeros = jnp.zeros((rpt_zero, d), jnp.float32)

    partial = _sc_segment_sum(src3, dst3, h, zeros, n_acc, rpt_zero, rpt_out, nch)

    blk = 1000
    grid = n // blk
    out = pl.pallas_call(
        _tc_body,
        grid=(grid,),
        in_specs=[
            pl.BlockSpec((blk, d), lambda i: (i, 0)),
            pl.BlockSpec((NC, blk, d), lambda i: (0, i, 0)),
            pl.BlockSpec((d, o), lambda i: (0, 0)),
            pl.BlockSpec((d, o), lambda i: (0, 0)),
            pl.BlockSpec((1, o), lambda i: (0, 0)),
            pl.BlockSpec((1, o), lambda i: (0, 0)),
            pl.BlockSpec((1, 2 * o), lambda i: (0, 0)),
            pl.BlockSpec((1, 2 * o), lambda i: (0, 0)),
        ],
        out_specs=pl.BlockSpec((blk, 2 * o), lambda i: (i, 0)),
        out_shape=jax.ShapeDtypeStruct((n, 2 * o), jnp.float32),
    )(h, partial, W_self, W_neigh, b_self.reshape(1, o), b_neigh.reshape(1, o),
      gamma.reshape(1, 2 * o), beta.reshape(1, 2 * o))
    return out


# speedup vs baseline: 7.0969x; 7.0969x over previous
import jax
import jax.numpy as jnp
from jax import lax
from jax.experimental import pallas as pl
from jax.experimental.pallas import tpu as pltpu
from jax.experimental.pallas import tpu_sc as plsc

NC = 2
NS = 16
CH = 128


def _sc_segment_sum(src3, dst3, h2, zeros, n_acc, rpt, nch, hd):
    mesh = plsc.VectorSubcoreMesh(
        core_axis_name="c", subcore_axis_name="s", num_cores=NC, num_subcores=NS
    )

    def body(src_hbm, dst_hbm, h_hbm, zeros_hbm, out_hbm,
             src_v, dst_v, buf0, buf1, acc, sem0, sem1):
        c = lax.axis_index("c")
        s = lax.axis_index("s")
        my_h = h_hbm.at[c]

        pltpu.sync_copy(src_hbm.at[s], src_v)
        pltpu.sync_copy(dst_hbm.at[s], dst_v)

        pltpu.async_copy(my_h.at[src_v.at[0]], buf0, sem0)
        pltpu.async_copy(my_h.at[src_v.at[1]], buf1, sem1)

        pltpu.sync_copy(zeros_hbm, acc.at[pl.ds(s * rpt, rpt)])
        plsc.subcore_barrier()

        bufs = (buf0, buf1)
        sems = (sem0, sem1)

        def chunk_pair(k, carry):
            for b in range(2):
                j = 2 * k + b
                pltpu.make_async_copy(my_h.at[src_v.at[j]], bufs[b], sems[b]).wait()
                pltpu.sync_copy(bufs[b], acc.at[dst_v.at[j]], add=True)
                pltpu.async_copy(my_h.at[src_v.at[j + 2]], bufs[b], sems[b])
            return carry

        lax.fori_loop(0, nch // 2 - 1, chunk_pair, 0)

        for b in range(2):
            j = nch - 2 + b
            pltpu.make_async_copy(my_h.at[src_v.at[j]], bufs[b], sems[b]).wait()
            pltpu.sync_copy(bufs[b], acc.at[dst_v.at[j]], add=True)

        plsc.subcore_barrier()
        pltpu.sync_copy(acc.at[pl.ds(s * rpt, rpt)],
                        out_hbm.at[c, pl.ds(s * rpt, rpt)])

    fn = pl.kernel(
        body,
        out_type=jax.ShapeDtypeStruct((NC, n_acc, hd), jnp.float32),
        mesh=mesh,
        scratch_types=[
            pltpu.VMEM((nch, CH), jnp.int32),
            pltpu.VMEM((nch, CH), jnp.int32),
            pltpu.VMEM((CH, hd), jnp.float32),
            pltpu.VMEM((CH, hd), jnp.float32),
            pltpu.VMEM_SHARED((n_acc, hd), jnp.float32),
            pltpu.SemaphoreType.DMA,
            pltpu.SemaphoreType.DMA,
        ],
        compiler_params=pltpu.CompilerParams(use_tc_tiling_on_sc=False),
    )
    return fn(src3, dst3, h2, zeros)


def _tc_body(h_ref, p_ref, ws_ref, wn_ref, bs_ref, bn_ref, g_ref, be_ref, out_ref):
    x = h_ref[...]
    p = jnp.concatenate([p_ref[0], p_ref[1]], axis=1)
    sh = jnp.dot(x, ws_ref[...], preferred_element_type=jnp.float32) + bs_ref[...]
    nh = jnp.dot(p, wn_ref[...], preferred_element_type=jnp.float32) + bn_ref[...]
    cat = jnp.concatenate([sh, nh], axis=1)
    mu = jnp.mean(cat, axis=1, keepdims=True)
    var = jnp.mean((cat - mu) * (cat - mu), axis=1, keepdims=True)
    out_ref[...] = (cat - mu) * lax.rsqrt(var + 1e-5) * g_ref[...] + be_ref[...]


def kernel(edge_index, h, W_self, b_self, W_neigh, b_neigh, gamma, beta):
    n, d = h.shape
    o = W_self.shape[1]
    e = edge_index.shape[1]
    hd = d // NC

    nch = -(-e // (NS * CH))
    nch += nch % 2
    e_pad = NS * CH * nch
    dst = edge_index[0]
    src = edge_index[1]
    src_p = jnp.concatenate([src, jnp.zeros((e_pad - e,), jnp.int32)])
    dst_p = jnp.concatenate([dst, jnp.full((e_pad - e,), n, jnp.int32)])
    src3 = src_p.reshape(NS, nch, CH)
    dst3 = dst_p.reshape(NS, nch, CH)
    h2 = jnp.transpose(h.reshape(n, NC, hd), (1, 0, 2))

    rpt = 8 * (-(-(n + 1) // (NS * 8)))
    n_acc = NS * rpt
    zeros = jnp.zeros((rpt, hd), jnp.float32)

    partial = _sc_segment_sum(src3, dst3, h2, zeros, n_acc, rpt, nch, hd)

    blk = 1000
    grid = n // blk
    out = pl.pallas_call(
        _tc_body,
        grid=(grid,),
        in_specs=[
            pl.BlockSpec((blk, d), lambda i: (i, 0)),
            pl.BlockSpec((NC, blk, hd), lambda i: (0, i, 0)),
            pl.BlockSpec((d, o), lambda i: (0, 0)),
            pl.BlockSpec((d, o), lambda i: (0, 0)),
            pl.BlockSpec((1, o), lambda i: (0, 0)),
            pl.BlockSpec((1, o), lambda i: (0, 0)),
            pl.BlockSpec((1, 2 * o), lambda i: (0, 0)),
            pl.BlockSpec((1, 2 * o), lambda i: (0, 0)),
        ],
        out_specs=pl.BlockSpec((blk, 2 * o), lambda i: (i, 0)),
        out_shape=jax.ShapeDtypeStruct((n, 2 * o), jnp.float32),
    )(h, partial, W_self, W_neigh, b_self.reshape(1, o), b_neigh.reshape(1, o),
      gamma.reshape(1, 2 * o), beta.reshape(1, 2 * o))
    return out
